# baseline (speedup 1.0000x reference)
import numpy as np
import jax
import jax.numpy as jnp
from jax import lax
from jax.experimental import pallas as pl
from jax.experimental.pallas import tpu as pltpu
from jax.experimental.pallas import tpu_sc as plsc

F_CAM = float((640 / 2.0) / np.tan(np.deg2rad(79.0 / 2.0)))
COS90 = float(np.cos(np.pi / 2.0))
SIN90 = float(np.sin(np.pi / 2.0))
S239 = float(2.0 / 239.0)
INV_F = float(1.0 / ((640 / 2.0) / np.tan(np.deg2rad(79.0 / 2.0))))
F32 = jnp.float32
I32 = jnp.int32


def _floorparts(a):
    ti = a.astype(I32)
    fl = jnp.where(a < ti.astype(F32), ti - 1, ti)
    return fl, a - fl.astype(F32)


def _r16(x):
    u = plsc.bitcast(x, I32)
    rb = (u >> 16) & 1
    return plsc.bitcast((u + 32767 + rb) & (-65536), F32)


def _mask_f(m):
    return jnp.where(m, jnp.full((16,), 1.0, F32), jnp.full((16,), 0.0, F32))


def _sc_body(depth, params, maps, fp_out, mp_out,
             dbuf, planes, pv, slab, tmp, rowb, rotw, mrow, s16i, s16f,
             sh_all, sh_red, sh_rot):
    cid = lax.axis_index("c")
    sid = lax.axis_index("s")

    @pl.when(cid == 0)
    def _work():
        t = sid
        pltpu.sync_copy(params, pv)
        iota = lax.iota(I32, 16)
        ah = pv[pl.ds(0, 16)]
        ct = pv[pl.ds(16, 16)]
        sn = pv[pl.ds(32, 16)]
        tx = pv[pl.ds(48, 16)]
        ty = pv[pl.ds(64, 16)]
        zeros16 = jnp.full((16,), 0.0, F32)
        ct16 = _r16(ct)
        sn16 = _r16(sn)
        tx16 = _r16(tx)
        ty16 = _r16(ty)

        def _zero(i, c):
            planes[pl.ds(i * 16, 16)] = zeros16
            return c
        lax.fori_loop(0, 1280, _zero, 0)

        def _row_a(ri, c):
            r_img = t * 30 + ri
            pltpu.sync_copy(depth.at[pl.ds(r_img * 640, 640)], dbuf)
            yfac = (240.0 - r_img.astype(F32)) * INV_F

            def _vec_a(v, c2):
                c0 = v * 16
                d = dbuf[pl.ds(c0, 16)] * 1000.0
                xs = (c0 + iota).astype(F32)
                X = (xs - 320.0) * INV_F * d
                Yv = yfac * d
                Zw = Yv + ah
                Xr = X * COS90 - d * SIN90 + 250.0
                Yr = X * SIN90 + d * COS90
                px = ((Xr * 0.2 - 50.0) * 0.01 * 2.0 + 1.0) * 0.5 * 99.0
                py = ((Yr * 0.2 - 50.0) * 0.01 * 2.0 + 1.0) * 0.5 * 99.0
                pz = ((Zw * 0.2 - 28.0) * (1.0 / 88.0) * 2.0 + 1.0) * 0.5 * 87.0
                ix, rx = _floorparts(px)
                iy, ry = _floorparts(py)
                iz, rz = _floorparts(pz)

                def _zin(z, lo, hi):
                    return _mask_f((z >= lo) & (z < hi))
                zall = (1.0 - rz) * _zin(iz, 0, 88) + rz * _zin(iz + 1, 0, 88)
                zag = (1.0 - rz) * _zin(iz, 21, 33) + rz * _zin(iz + 1, 21, 33)

                base = iy * 100 + ix
                ks, ixs, iys, rxs, rys, zags, zalls = lax.sort(
                    (base, ix, iy, rx, ry, zag, zall),
                    dimension=0, num_keys=1)
                s16i[pl.ds(0, 16)] = ks
                prev = plsc.load_gather(s16i, [jnp.maximum(iota - 1, 0)])
                nxt = plsc.load_gather(s16i, [jnp.minimum(iota + 1, 15)])
                head = (iota == 0) | (ks != prev)
                tail = (iota == 15) | (ks != nxt)
                hl = plsc.cummax(jnp.where(head, iota, 0))
                hlm1 = jnp.maximum(hl - 1, 0)
                hpos = hl >= 1

                for dx in (0, 1):
                    cx = ixs + dx if dx else ixs
                    wx = rxs if dx else 1.0 - rxs
                    for dy in (0, 1):
                        cy = iys + dy if dy else iys
                        wy = rys if dy else 1.0 - rys
                        m = (cx >= 0) & (cx < 100) & (cy >= 0) & (cy < 100)
                        mf = _mask_f(m)
                        w0 = wx * wy * mf * zags
                        w1 = wx * wy * mf * zalls
                        cum0 = plsc.cumsum(w0)
                        cum1 = plsc.cumsum(w1)
                        s16f[pl.ds(0, 16)] = cum0
                        s16f[pl.ds(16, 16)] = cum1
                        b0 = plsc.load_gather(s16f, [hlm1])
                        b1 = plsc.load_gather(s16f, [hlm1 + 16])
                        seg0 = cum0 - jnp.where(hpos, b0, 0.0)
                        seg1 = cum1 - jnp.where(hpos, b1, 0.0)
                        idx = (jnp.clip(cy, 0, 99) * 100
                               + jnp.clip(cx, 0, 99))
                        idx0 = jnp.where(m, idx, 20000 + iota)
                        idx1 = jnp.where(m, idx + 10000, 20000 + iota)
                        plsc.addupdate_scatter(planes, [idx0], seg0,
                                               mask=tail)
                        plsc.addupdate_scatter(planes, [idx1], seg1,
                                               mask=tail)
                return c2
            lax.fori_loop(0, 40, _vec_a, 0)
            return c
        lax.fori_loop(0, 30, _row_a, 0)

        pltpu.sync_copy(planes, sh_all.at[pl.ds(t * 20480, 20480)])
        plsc.subcore_barrier()
        off = t * 1280
        pltpu.sync_copy(sh_all.at[pl.ds(off, 1280)], slab)

        def _acc(k, c):
            pltpu.sync_copy(sh_all.at[pl.ds(k * 20480 + off, 1280)], tmp)

            def _add(i, c2):
                s = pl.ds(i * 16, 16)
                slab[s] = slab[s] + tmp[s]
                return c2
            lax.fori_loop(0, 80, _add, 0)
            return c
        lax.fori_loop(1, 16, _acc, 0)
        pltpu.sync_copy(slab, sh_red.at[pl.ds(off, 1280)])
        plsc.subcore_barrier()
        pltpu.sync_copy(sh_red, planes)

        def _clip(i, c):
            s = pl.ds(i * 16, 16)
            planes[s] = jnp.minimum(jnp.maximum(planes[s], 0.0), 1.0)
            return c
        lax.fori_loop(0, 1280, _clip, 0)

        @pl.when(t == 0)
        def _fp():
            pltpu.sync_copy(planes.at[pl.ds(0, 10000)], fp_out)

        def _row_b(rl, c):
            r = t * 15 + rl
            gy = -1.0 + r.astype(F32) * S239

            gy16 = _r16(jnp.full((16,), 0.0, F32) + gy)

            def _vec_b(v, c2):
                cc = v * 16 + iota
                gx16 = _r16(-1.0 + cc.astype(F32) * S239)
                x = (gx16 * ct16 - gy16 * sn16 + 1.0) * 119.5
                y = (gx16 * sn16 + gy16 * ct16 + 1.0) * 119.5
                x0, fxw = _floorparts(x)
                y0, fyw = _floorparts(y)
                acc0 = zeros16
                acc1 = zeros16
                for a in (0, 1):
                    xi = x0 + a if a else x0
                    wxv = fxw if a else 1.0 - fxw
                    for b in (0, 1):
                        yi = y0 + b if b else y0
                        wyv = fyw if b else 1.0 - fyw
                        m = ((yi >= 120) & (yi <= 219)
                             & (xi >= 70) & (xi <= 169))
                        pi = (jnp.clip(yi - 120, 0, 99) * 100
                              + jnp.clip(xi - 70, 0, 99))
                        w = wxv * wyv * _mask_f(m)
                        acc0 = acc0 + w * plsc.load_gather(planes, [pi])
                        acc1 = acc1 + w * plsc.load_gather(planes,
                                                           [pi + 10000])
                rowb[pl.ds(v * 16, 16)] = acc0
                rowb[pl.ds(240 + v * 16, 16)] = acc1
                return c2
            lax.fori_loop(0, 15, _vec_b, 0)
            pltpu.sync_copy(rowb.at[pl.ds(0, 240)],
                            sh_rot.at[pl.ds(r * 240, 240)])
            pltpu.sync_copy(rowb.at[pl.ds(240, 240)],
                            sh_rot.at[pl.ds(57600 + r * 240, 240)])
            return c
        lax.fori_loop(0, 15, _row_b, 0)
        plsc.subcore_barrier()

        r0 = t * 15
        y0v = (ty16 + _r16(jnp.full((16,), 0.0, F32)
                           + (-1.0 + r0.astype(F32) * S239)) + 1.0) * 119.5
        ys0 = jnp.min(y0v)
        si = ys0.astype(I32)
        si = jnp.where(ys0 < si.astype(F32), si - 1, si)
        start_c = jnp.clip(si - 1, 0, 240 - 24)
        pltpu.sync_copy(sh_rot.at[pl.ds(start_c * 240, 5760)],
                        rotw.at[pl.ds(0, 5760)])
        pltpu.sync_copy(sh_rot.at[pl.ds(57600 + start_c * 240, 5760)],
                        rotw.at[pl.ds(5760, 5760)])

        def _row_c(rl, c):
            r = r0 + rl
            gy16 = _r16(jnp.full((16,), -1.0, F32) + r.astype(F32) * S239)
            yrow = (gy16 + ty16 + 1.0) * 119.5
            y0i, fyw = _floorparts(yrow)

            def _vec_c(v, c2):
                cc = v * 16 + iota
                gx16 = _r16(-1.0 + cc.astype(F32) * S239)
                x = (gx16 + tx16 + 1.0) * 119.5
                x0i, fxw = _floorparts(x)
                acc0 = zeros16
                acc1 = zeros16
                for a in (0, 1):
                    xi = x0i + a if a else x0i
                    wxv = fxw if a else 1.0 - fxw
                    xm = (xi >= 0) & (xi <= 239)
                    xc = jnp.clip(xi, 0, 239)
                    for b in (0, 1):
                        yi = y0i + b if b else y0i
                        wyv = fyw if b else 1.0 - fyw
                        m = xm & (yi >= 0) & (yi <= 239)
                        rloc = jnp.clip(yi - start_c, 0, 23)
                        w = wxv * wyv * _mask_f(m)
                        gi = rloc * 240 + xc
                        acc0 = acc0 + w * plsc.load_gather(rotw, [gi])
                        acc1 = acc1 + w * plsc.load_gather(rotw, [gi + 5760])
                rowb[pl.ds(v * 16, 16)] = acc0
                rowb[pl.ds(240 + v * 16, 16)] = acc1
                return c2
            lax.fori_loop(0, 15, _vec_c, 0)

            pltpu.sync_copy(maps.at[pl.ds(r * 240, 240)], mrow)

            def _mx0(i, c2):
                s = pl.ds(i * 16, 16)
                rowb[s] = jnp.maximum(rowb[s], mrow[s])
                return c2
            lax.fori_loop(0, 15, _mx0, 0)
            pltpu.sync_copy(rowb.at[pl.ds(0, 240)],
                            mp_out.at[pl.ds(r * 240, 240)])

            pltpu.sync_copy(maps.at[pl.ds(57600 + r * 240, 240)], mrow)

            def _mx1b(i, c2):
                sm = pl.ds(i * 16, 16)
                sr = pl.ds(240 + i * 16, 16)
                rowb[sr] = jnp.maximum(rowb[sr], mrow[sm])
                return c2
            lax.fori_loop(0, 15, _mx1b, 0)
            pltpu.sync_copy(rowb.at[pl.ds(240, 240)],
                            mp_out.at[pl.ds(57600 + r * 240, 240)])

            pltpu.sync_copy(maps.at[pl.ds(115200 + r * 240, 240)], mrow)
            pltpu.sync_copy(mrow, mp_out.at[pl.ds(115200 + r * 240, 240)])
            pltpu.sync_copy(maps.at[pl.ds(172800 + r * 240, 240)], mrow)
            pltpu.sync_copy(mrow, mp_out.at[pl.ds(172800 + r * 240, 240)])
            return c
        lax.fori_loop(0, 15, _row_c, 0)


def _make_sc_call():
    mesh = plsc.VectorSubcoreMesh(core_axis_name="c", subcore_axis_name="s")
    return pl.kernel(
        _sc_body,
        mesh=mesh,
        compiler_params=pltpu.CompilerParams(needs_layout_passes=False),
        out_type=(
            jax.ShapeDtypeStruct((10000,), F32),
            jax.ShapeDtypeStruct((230400,), F32),
        ),
        scratch_types=[
            pltpu.VMEM((640,), F32),
            pltpu.VMEM((20480,), F32),
            pltpu.VMEM((80,), F32),
            pltpu.VMEM((1280,), F32),
            pltpu.VMEM((1280,), F32),
            pltpu.VMEM((480,), F32),
            pltpu.VMEM((11520,), F32),
            pltpu.VMEM((240,), F32),
            pltpu.VMEM((16,), I32),
            pltpu.VMEM((32,), F32),
            pltpu.VMEM_SHARED((327680,), F32),
            pltpu.VMEM_SHARED((20480,), F32),
            pltpu.VMEM_SHARED((115200,), F32),
        ],
    )


def kernel(obs, pose_obs, maps_last, poses_last, agent_heights):
    depth = obs[0, 3].reshape(-1)
    pose = poses_last[0]
    rel = pose_obs[0]
    o_rad = pose[2] * float(np.pi / 180.0)
    yp = pose[1] + rel[1] * jnp.sin(o_rad) + rel[0] * jnp.cos(o_rad)
    xp = pose[0] + rel[1] * jnp.cos(o_rad) - rel[0] * jnp.sin(o_rad)
    o = pose[2] + rel[2] * 57.29577951308232
    o = jnp.fmod(o - 180.0, 360.0) + 180.0
    o = jnp.fmod(o + 180.0, 360.0) - 180.0
    current_poses = jnp.stack([xp, yp, o])[None]
    st0 = -(yp * 100.0 / 5.0 - 120.0) / 120.0
    st1 = -(xp * 100.0 / 5.0 - 120.0) / 120.0
    st2 = 90.0 - o
    tr = st2 * float(np.pi / 180.0)
    ctv = jnp.cos(tr)
    snv = jnp.sin(tr)
    ah = 88.0 * agent_heights[0]
    params = (jnp.stack([ah, ctv, snv, st0, st1]).astype(F32)[:, None]
              * jnp.ones((1, 16), F32)).reshape(-1)
    fp_flat, mp = _sc_call(depth, params, maps_last.reshape(-1))
    fp_map_pred = fp_flat.reshape(1, 1, 100, 100)
    return fp_map_pred, mp.reshape(1, 4, 240, 240), poses_last, current_poses


_sc_call = _make_sc_call()


# speedup vs baseline: 1.2861x; 1.2861x over previous
import numpy as np
import jax
import jax.numpy as jnp
from jax import lax
from jax.experimental import pallas as pl
from jax.experimental.pallas import tpu as pltpu
from jax.experimental.pallas import tpu_sc as plsc

F_CAM = float((640 / 2.0) / np.tan(np.deg2rad(79.0 / 2.0)))
COS90 = float(np.cos(np.pi / 2.0))
SIN90 = float(np.sin(np.pi / 2.0))
S239 = float(2.0 / 239.0)
INV_F = float(1.0 / ((640 / 2.0) / np.tan(np.deg2rad(79.0 / 2.0))))
F32 = jnp.float32
I32 = jnp.int32


def _floorparts(a):
    ti = a.astype(I32)
    fl = jnp.where(a < ti.astype(F32), ti - 1, ti)
    return fl, a - fl.astype(F32)


def _r16(x):
    u = plsc.bitcast(x, I32)
    rb = (u >> 16) & 1
    return plsc.bitcast((u + 32767 + rb) & (-65536), F32)


def _mask_f(m):
    return jnp.where(m, jnp.full((16,), 1.0, F32), jnp.full((16,), 0.0, F32))


def _sc_body(depth, params, maps, fp_out, mp_out,
             dbuf, planes, pv, slab, tmp, rowb, rotw, mrow,
             sh_all, sh_red, sh_rot):
    cid = lax.axis_index("c")
    sid = lax.axis_index("s")

    @pl.when(cid == 0)
    def _work():
        t = sid
        pltpu.sync_copy(params, pv)
        iota = lax.iota(I32, 16)
        ah = pv[pl.ds(0, 16)]
        ct = pv[pl.ds(16, 16)]
        sn = pv[pl.ds(32, 16)]
        tx = pv[pl.ds(48, 16)]
        ty = pv[pl.ds(64, 16)]
        zeros16 = jnp.full((16,), 0.0, F32)
        ct16 = _r16(ct)
        sn16 = _r16(sn)
        tx16 = _r16(tx)
        ty16 = _r16(ty)

        def _zero(i, c):
            planes[pl.ds(i * 16, 16)] = zeros16
            return c
        lax.fori_loop(0, 1280, _zero, 0)

        def _row_a(ri, c):
            r_img = t * 30 + ri
            pltpu.sync_copy(depth.at[pl.ds(r_img * 640, 640)], dbuf)
            yfac = (240.0 - r_img.astype(F32)) * INV_F

            def _vec_a(v, c2):
                c0 = v * 16
                d = dbuf[pl.ds(c0, 16)] * 1000.0
                xs = (c0 + iota).astype(F32)
                X = (xs - 320.0) * INV_F * d
                Yv = yfac * d
                Zw = Yv + ah
                Xr = X * COS90 - d * SIN90 + 250.0
                Yr = X * SIN90 + d * COS90
                px = ((Xr * 0.2 - 50.0) * 0.01 * 2.0 + 1.0) * 0.5 * 99.0
                py = ((Yr * 0.2 - 50.0) * 0.01 * 2.0 + 1.0) * 0.5 * 99.0
                pz = ((Zw * 0.2 - 28.0) * (1.0 / 88.0) * 2.0 + 1.0) * 0.5 * 87.0
                ix, rx = _floorparts(px)
                iy, ry = _floorparts(py)
                iz, rz = _floorparts(pz)

                def _zin(z, lo, hi):
                    return _mask_f((z >= lo) & (z < hi))
                zall = (1.0 - rz) * _zin(iz, 0, 88) + rz * _zin(iz + 1, 0, 88)
                zag = (1.0 - rz) * _zin(iz, 21, 33) + rz * _zin(iz + 1, 21, 33)

                for dx in (0, 1):
                    cx = ix + dx if dx else ix
                    wx = rx if dx else 1.0 - rx
                    for dy in (0, 1):
                        cy = iy + dy if dy else iy
                        wy = ry if dy else 1.0 - ry
                        m = (cx >= 0) & (cx < 100) & (cy >= 0) & (cy < 100)
                        idx = (jnp.clip(cy, 0, 99) * 100
                               + jnp.clip(cx, 0, 99))
                        w = wx * wy
                        plsc.addupdate_scatter(planes, [idx], w * zag, mask=m)
                        plsc.addupdate_scatter(planes, [idx + 10000],
                                               w * zall, mask=m)
                return c2
            lax.fori_loop(0, 40, _vec_a, 0)
            return c
        lax.fori_loop(0, 30, _row_a, 0)

        pltpu.sync_copy(planes, sh_all.at[pl.ds(t * 20480, 20480)])
        plsc.subcore_barrier()
        off = t * 1280
        pltpu.sync_copy(sh_all.at[pl.ds(off, 1280)], slab)

        def _acc(k, c):
            pltpu.sync_copy(sh_all.at[pl.ds(k * 20480 + off, 1280)], tmp)

            def _add(i, c2):
                s = pl.ds(i * 16, 16)
                slab[s] = slab[s] + tmp[s]
                return c2
            lax.fori_loop(0, 80, _add, 0)
            return c
        lax.fori_loop(1, 16, _acc, 0)
        pltpu.sync_copy(slab, sh_red.at[pl.ds(off, 1280)])
        plsc.subcore_barrier()
        pltpu.sync_copy(sh_red, planes)

        def _clip(i, c):
            s = pl.ds(i * 16, 16)
            planes[s] = jnp.minimum(jnp.maximum(planes[s], 0.0), 1.0)
            return c
        lax.fori_loop(0, 1280, _clip, 0)

        @pl.when(t == 0)
        def _fp():
            pltpu.sync_copy(planes.at[pl.ds(0, 10000)], fp_out)

        def _row_b(rl, c):
            r = t * 15 + rl
            gy = -1.0 + r.astype(F32) * S239

            gy16 = _r16(jnp.full((16,), 0.0, F32) + gy)

            def _vec_b(v, c2):
                cc = v * 16 + iota
                gx16 = _r16(-1.0 + cc.astype(F32) * S239)
                x = (gx16 * ct16 - gy16 * sn16 + 1.0) * 119.5
                y = (gx16 * sn16 + gy16 * ct16 + 1.0) * 119.5
                x0, fxw = _floorparts(x)
                y0, fyw = _floorparts(y)
                acc0 = zeros16
                acc1 = zeros16
                for a in (0, 1):
                    xi = x0 + a if a else x0
                    wxv = fxw if a else 1.0 - fxw
                    for b in (0, 1):
                        yi = y0 + b if b else y0
                        wyv = fyw if b else 1.0 - fyw
                        m = ((yi >= 120) & (yi <= 219)
                             & (xi >= 70) & (xi <= 169))
                        pi = (jnp.clip(yi - 120, 0, 99) * 100
                              + jnp.clip(xi - 70, 0, 99))
                        w = wxv * wyv * _mask_f(m)
                        acc0 = acc0 + w * plsc.load_gather(planes, [pi])
                        acc1 = acc1 + w * plsc.load_gather(planes,
                                                           [pi + 10000])
                rowb[pl.ds(v * 16, 16)] = acc0
                rowb[pl.ds(240 + v * 16, 16)] = acc1
                return c2
            lax.fori_loop(0, 15, _vec_b, 0)
            pltpu.sync_copy(rowb.at[pl.ds(0, 240)],
                            sh_rot.at[pl.ds(r * 240, 240)])
            pltpu.sync_copy(rowb.at[pl.ds(240, 240)],
                            sh_rot.at[pl.ds(57600 + r * 240, 240)])
            return c
        lax.fori_loop(0, 15, _row_b, 0)
        plsc.subcore_barrier()

        r0 = t * 15
        y0v = (ty16 + _r16(jnp.full((16,), 0.0, F32)
                           + (-1.0 + r0.astype(F32) * S239)) + 1.0) * 119.5
        ys0 = jnp.min(y0v)
        si = ys0.astype(I32)
        si = jnp.where(ys0 < si.astype(F32), si - 1, si)
        start_c = jnp.clip(si - 1, 0, 240 - 24)
        pltpu.sync_copy(sh_rot.at[pl.ds(start_c * 240, 5760)],
                        rotw.at[pl.ds(0, 5760)])
        pltpu.sync_copy(sh_rot.at[pl.ds(57600 + start_c * 240, 5760)],
                        rotw.at[pl.ds(5760, 5760)])

        def _row_c(rl, c):
            r = r0 + rl
            gy16 = _r16(jnp.full((16,), -1.0, F32) + r.astype(F32) * S239)
            yrow = (gy16 + ty16 + 1.0) * 119.5
            y0i, fyw = _floorparts(yrow)

            def _vec_c(v, c2):
                cc = v * 16 + iota
                gx16 = _r16(-1.0 + cc.astype(F32) * S239)
                x = (gx16 + tx16 + 1.0) * 119.5
                x0i, fxw = _floorparts(x)
                acc0 = zeros16
                acc1 = zeros16
                for a in (0, 1):
                    xi = x0i + a if a else x0i
                    wxv = fxw if a else 1.0 - fxw
                    xm = (xi >= 0) & (xi <= 239)
                    xc = jnp.clip(xi, 0, 239)
                    for b in (0, 1):
                        yi = y0i + b if b else y0i
                        wyv = fyw if b else 1.0 - fyw
                        m = xm & (yi >= 0) & (yi <= 239)
                        rloc = jnp.clip(yi - start_c, 0, 23)
                        w = wxv * wyv * _mask_f(m)
                        gi = rloc * 240 + xc
                        acc0 = acc0 + w * plsc.load_gather(rotw, [gi])
                        acc1 = acc1 + w * plsc.load_gather(rotw, [gi + 5760])
                rowb[pl.ds(v * 16, 16)] = acc0
                rowb[pl.ds(240 + v * 16, 16)] = acc1
                return c2
            lax.fori_loop(0, 15, _vec_c, 0)

            pltpu.sync_copy(maps.at[pl.ds(r * 240, 240)], mrow)

            def _mx0(i, c2):
                s = pl.ds(i * 16, 16)
                rowb[s] = jnp.maximum(rowb[s], mrow[s])
                return c2
            lax.fori_loop(0, 15, _mx0, 0)
            pltpu.sync_copy(rowb.at[pl.ds(0, 240)],
                            mp_out.at[pl.ds(r * 240, 240)])

            pltpu.sync_copy(maps.at[pl.ds(57600 + r * 240, 240)], mrow)

            def _mx1b(i, c2):
                sm = pl.ds(i * 16, 16)
                sr = pl.ds(240 + i * 16, 16)
                rowb[sr] = jnp.maximum(rowb[sr], mrow[sm])
                return c2
            lax.fori_loop(0, 15, _mx1b, 0)
            pltpu.sync_copy(rowb.at[pl.ds(240, 240)],
                            mp_out.at[pl.ds(57600 + r * 240, 240)])

            pltpu.sync_copy(maps.at[pl.ds(115200 + r * 240, 240)], mrow)
            pltpu.sync_copy(mrow, mp_out.at[pl.ds(115200 + r * 240, 240)])
            pltpu.sync_copy(maps.at[pl.ds(172800 + r * 240, 240)], mrow)
            pltpu.sync_copy(mrow, mp_out.at[pl.ds(172800 + r * 240, 240)])
            return c
        lax.fori_loop(0, 15, _row_c, 0)


def _make_sc_call():
    mesh = plsc.VectorSubcoreMesh(core_axis_name="c", subcore_axis_name="s")
    return pl.kernel(
        _sc_body,
        mesh=mesh,
        compiler_params=pltpu.CompilerParams(needs_layout_passes=False),
        out_type=(
            jax.ShapeDtypeStruct((10000,), F32),
            jax.ShapeDtypeStruct((230400,), F32),
        ),
        scratch_types=[
            pltpu.VMEM((640,), F32),
            pltpu.VMEM((20480,), F32),
            pltpu.VMEM((80,), F32),
            pltpu.VMEM((1280,), F32),
            pltpu.VMEM((1280,), F32),
            pltpu.VMEM((480,), F32),
            pltpu.VMEM((11520,), F32),
            pltpu.VMEM((240,), F32),
            pltpu.VMEM_SHARED((327680,), F32),
            pltpu.VMEM_SHARED((20480,), F32),
            pltpu.VMEM_SHARED((115200,), F32),
        ],
    )


def kernel(obs, pose_obs, maps_last, poses_last, agent_heights):
    depth = obs[0, 3].reshape(-1)
    pose = poses_last[0]
    rel = pose_obs[0]
    o_rad = pose[2] * float(np.pi / 180.0)
    yp = pose[1] + rel[1] * jnp.sin(o_rad) + rel[0] * jnp.cos(o_rad)
    xp = pose[0] + rel[1] * jnp.cos(o_rad) - rel[0] * jnp.sin(o_rad)
    o = pose[2] + rel[2] * 57.29577951308232
    o = jnp.fmod(o - 180.0, 360.0) + 180.0
    o = jnp.fmod(o + 180.0, 360.0) - 180.0
    current_poses = jnp.stack([xp, yp, o])[None]
    st0 = -(yp * 100.0 / 5.0 - 120.0) / 120.0
    st1 = -(xp * 100.0 / 5.0 - 120.0) / 120.0
    st2 = 90.0 - o
    tr = st2 * float(np.pi / 180.0)
    ctv = jnp.cos(tr)
    snv = jnp.sin(tr)
    ah = 88.0 * agent_heights[0]
    params = (jnp.stack([ah, ctv, snv, st0, st1]).astype(F32)[:, None]
              * jnp.ones((1, 16), F32)).reshape(-1)
    fp_flat, mp = _sc_call(depth, params, maps_last.reshape(-1))
    fp_map_pred = fp_flat.reshape(1, 1, 100, 100)
    return fp_map_pred, mp.reshape(1, 4, 240, 240), poses_last, current_poses


_sc_call = _make_sc_call()


# speedup vs baseline: 1.7501x; 1.3608x over previous
import numpy as np
import jax
import jax.numpy as jnp
from jax import lax
from jax.experimental import pallas as pl
from jax.experimental.pallas import tpu as pltpu
from jax.experimental.pallas import tpu_sc as plsc

F_CAM = float((640 / 2.0) / np.tan(np.deg2rad(79.0 / 2.0)))
COS90 = float(np.cos(np.pi / 2.0))
SIN90 = float(np.sin(np.pi / 2.0))
S239 = float(2.0 / 239.0)
INV_F = float(1.0 / F_CAM)
F32 = jnp.float32
I32 = jnp.int32


def _floorparts(a):
    ti = a.astype(I32)
    fl = jnp.where(a < ti.astype(F32), ti - 1, ti)
    return fl, a - fl.astype(F32)


def _r16(x):
    u = plsc.bitcast(x, I32)
    rb = (u >> 16) & 1
    return plsc.bitcast((u + 32767 + rb) & (-65536), F32)


def _mask_f(m):
    return jnp.where(m, jnp.full((16,), 1.0, F32), jnp.full((16,), 0.0, F32))


def _sc_body(depth, params, maps, fp_out, mp_out,
             dbuf, planes, pv, slab, tmp, rotbuf, rotw, mapsbuf,
             sh_all, sh_red, sh_rot):
    cid = lax.axis_index("c")
    sid = lax.axis_index("s")

    @pl.when(cid == 0)
    def _work():
        t = sid
        pltpu.sync_copy(params, pv)
        iota = lax.iota(I32, 16)
        ah = pv[pl.ds(0, 16)]
        ct = pv[pl.ds(16, 16)]
        sn = pv[pl.ds(32, 16)]
        tx = pv[pl.ds(48, 16)]
        ty = pv[pl.ds(64, 16)]
        zeros16 = jnp.full((16,), 0.0, F32)
        ct16 = _r16(ct)
        sn16 = _r16(sn)
        tx16 = _r16(tx)
        ty16 = _r16(ty)

        def _zero(i, c):
            planes[pl.ds(i * 16, 16)] = zeros16
            return c
        lax.fori_loop(0, 1280, _zero, 0)

        pltpu.sync_copy(depth.at[pl.ds(t * 19200, 19200)], dbuf)

        def _row_a(ri, c):
            r_img = t * 30 + ri
            yfac = (240.0 - r_img.astype(F32)) * INV_F
            roff = ri * 640

            def _vec_a(v, c2):
                c0 = v * 16
                d = dbuf[pl.ds(roff + c0, 16)] * 1000.0
                xs = (c0 + iota).astype(F32)
                X = (xs - 320.0) * INV_F * d
                Yv = yfac * d
                Zw = Yv + ah
                Xr = X * COS90 - d * SIN90 + 250.0
                Yr = X * SIN90 + d * COS90
                px = ((Xr * 0.2 - 50.0) * 0.01 * 2.0 + 1.0) * 0.5 * 99.0
                py = ((Yr * 0.2 - 50.0) * 0.01 * 2.0 + 1.0) * 0.5 * 99.0
                pz = ((Zw * 0.2 - 28.0) * (1.0 / 88.0) * 2.0 + 1.0) * 0.5 * 87.0
                ix, rx = _floorparts(px)
                iy, ry = _floorparts(py)
                iz, rz = _floorparts(pz)

                def _zin(z, lo, hi):
                    return _mask_f((z >= lo) & (z < hi))
                zall = (1.0 - rz) * _zin(iz, 0, 88) + rz * _zin(iz + 1, 0, 88)
                zag = (1.0 - rz) * _zin(iz, 21, 33) + rz * _zin(iz + 1, 21, 33)

                for dx in (0, 1):
                    cx = ix + dx if dx else ix
                    wx = rx if dx else 1.0 - rx
                    for dy in (0, 1):
                        cy = iy + dy if dy else iy
                        wy = ry if dy else 1.0 - ry
                        m = (cx >= 0) & (cx < 100) & (cy >= 0) & (cy < 100)
                        idx = (jnp.clip(cy, 0, 99) * 100
                               + jnp.clip(cx, 0, 99))
                        w = wx * wy
                        plsc.addupdate_scatter(planes, [idx], w * zag, mask=m)
                        plsc.addupdate_scatter(planes, [idx + 10000],
                                               w * zall, mask=m)
                return c2
            lax.fori_loop(0, 40, _vec_a, 0)
            return c
        lax.fori_loop(0, 30, _row_a, 0)

        pltpu.sync_copy(planes, sh_all.at[pl.ds(t * 20480, 20480)])
        plsc.subcore_barrier()
        off = t * 1280
        pltpu.sync_copy(sh_all.at[pl.ds(off, 1280)], slab)

        def _acc(k, c):
            pltpu.sync_copy(sh_all.at[pl.ds(k * 20480 + off, 1280)], tmp)

            def _add(i, c2):
                s = pl.ds(i * 16, 16)
                slab[s] = slab[s] + tmp[s]
                return c2
            lax.fori_loop(0, 80, _add, 0)
            return c
        lax.fori_loop(1, 16, _acc, 0)
        pltpu.sync_copy(slab, sh_red.at[pl.ds(off, 1280)])
        plsc.subcore_barrier()
        pltpu.sync_copy(sh_red, planes)

        def _clip(i, c):
            s = pl.ds(i * 16, 16)
            planes[s] = jnp.minimum(jnp.maximum(planes[s], 0.0), 1.0)
            return c
        lax.fori_loop(0, 1280, _clip, 0)

        @pl.when(t == 0)
        def _fp():
            pltpu.sync_copy(planes.at[pl.ds(0, 10000)], fp_out)

        r0 = t * 15

        def _row_b(rl, c):
            r = r0 + rl
            gy16 = _r16(jnp.full((16,), -1.0, F32) + r.astype(F32) * S239)

            def _vec_b(v, c2):
                cc = v * 16 + iota
                gx16 = _r16(-1.0 + cc.astype(F32) * S239)
                x = (gx16 * ct16 - gy16 * sn16 + 1.0) * 119.5
                y = (gx16 * sn16 + gy16 * ct16 + 1.0) * 119.5
                x0, fxw = _floorparts(x)
                y0, fyw = _floorparts(y)
                acc0 = zeros16
                acc1 = zeros16
                for a in (0, 1):
                    xi = x0 + a if a else x0
                    wxv = fxw if a else 1.0 - fxw
                    for b in (0, 1):
                        yi = y0 + b if b else y0
                        wyv = fyw if b else 1.0 - fyw
                        m = ((yi >= 120) & (yi <= 219)
                             & (xi >= 70) & (xi <= 169))
                        pi = (jnp.clip(yi - 120, 0, 99) * 100
                              + jnp.clip(xi - 70, 0, 99))
                        w = wxv * wyv * _mask_f(m)
                        acc0 = acc0 + w * plsc.load_gather(planes, [pi])
                        acc1 = acc1 + w * plsc.load_gather(planes,
                                                           [pi + 10000])
                rotbuf[pl.ds(rl * 240 + v * 16, 16)] = acc0
                rotbuf[pl.ds(3600 + rl * 240 + v * 16, 16)] = acc1
                return c2
            lax.fori_loop(0, 15, _vec_b, 0)
            return c
        lax.fori_loop(0, 15, _row_b, 0)
        pltpu.sync_copy(rotbuf.at[pl.ds(0, 3600)],
                        sh_rot.at[pl.ds(r0 * 240, 3600)])
        pltpu.sync_copy(rotbuf.at[pl.ds(3600, 3600)],
                        sh_rot.at[pl.ds(57600 + r0 * 240, 3600)])
        plsc.subcore_barrier()

        y0v = (ty16 + _r16(jnp.full((16,), 0.0, F32)
                           + (-1.0 + r0.astype(F32) * S239)) + 1.0) * 119.5
        ys0 = jnp.min(y0v)
        si = ys0.astype(I32)
        si = jnp.where(ys0 < si.astype(F32), si - 1, si)
        start_c = jnp.clip(si - 1, 0, 240 - 24)
        pltpu.sync_copy(sh_rot.at[pl.ds(start_c * 240, 5760)],
                        rotw.at[pl.ds(0, 5760)])
        pltpu.sync_copy(sh_rot.at[pl.ds(57600 + start_c * 240, 5760)],
                        rotw.at[pl.ds(5760, 5760)])

        def _row_c(rl, c):
            r = r0 + rl
            gy16 = _r16(jnp.full((16,), -1.0, F32) + r.astype(F32) * S239)
            yrow = (gy16 + ty16 + 1.0) * 119.5
            y0i, fyw = _floorparts(yrow)

            def _vec_c(v, c2):
                cc = v * 16 + iota
                gx16 = _r16(-1.0 + cc.astype(F32) * S239)
                x = (gx16 + tx16 + 1.0) * 119.5
                x0i, fxw = _floorparts(x)
                acc0 = zeros16
                acc1 = zeros16
                for a in (0, 1):
                    xi = x0i + a if a else x0i
                    wxv = fxw if a else 1.0 - fxw
                    xm = (xi >= 0) & (xi <= 239)
                    xc = jnp.clip(xi, 0, 239)
                    for b in (0, 1):
                        yi = y0i + b if b else y0i
                        wyv = fyw if b else 1.0 - fyw
                        m = xm & (yi >= 0) & (yi <= 239)
                        rloc = jnp.clip(yi - start_c, 0, 23)
                        w = wxv * wyv * _mask_f(m)
                        gi = rloc * 240 + xc
                        acc0 = acc0 + w * plsc.load_gather(rotw, [gi])
                        acc1 = acc1 + w * plsc.load_gather(rotw, [gi + 5760])
                rotbuf[pl.ds(rl * 240 + v * 16, 16)] = acc0
                rotbuf[pl.ds(3600 + rl * 240 + v * 16, 16)] = acc1
                return c2
            lax.fori_loop(0, 15, _vec_c, 0)
            return c
        lax.fori_loop(0, 15, _row_c, 0)

        for ch in (0, 1):
            pltpu.sync_copy(maps.at[pl.ds(ch * 57600 + r0 * 240, 3600)],
                            mapsbuf)
            coff = ch * 3600

            def _mx(i, c2, coff=coff):
                sr = pl.ds(coff + i * 16, 16)
                sm = pl.ds(i * 16, 16)
                rotbuf[sr] = jnp.maximum(rotbuf[sr], mapsbuf[sm])
                return c2
            lax.fori_loop(0, 225, _mx, 0)
            pltpu.sync_copy(rotbuf.at[pl.ds(coff, 3600)],
                            mp_out.at[pl.ds(ch * 57600 + r0 * 240, 3600)])

        for ch in (2, 3):
            pltpu.sync_copy(maps.at[pl.ds(ch * 57600 + r0 * 240, 3600)],
                            mapsbuf)
            pltpu.sync_copy(mapsbuf,
                            mp_out.at[pl.ds(ch * 57600 + r0 * 240, 3600)])


def _make_sc_call():
    mesh = plsc.VectorSubcoreMesh(core_axis_name="c", subcore_axis_name="s")
    return pl.kernel(
        _sc_body,
        mesh=mesh,
        compiler_params=pltpu.CompilerParams(needs_layout_passes=False),
        out_type=(
            jax.ShapeDtypeStruct((10000,), F32),
            jax.ShapeDtypeStruct((230400,), F32),
        ),
        scratch_types=[
            pltpu.VMEM((19200,), F32),
            pltpu.VMEM((20480,), F32),
            pltpu.VMEM((80,), F32),
            pltpu.VMEM((1280,), F32),
            pltpu.VMEM((1280,), F32),
            pltpu.VMEM((7200,), F32),
            pltpu.VMEM((11520,), F32),
            pltpu.VMEM((3600,), F32),
            pltpu.VMEM_SHARED((327680,), F32),
            pltpu.VMEM_SHARED((20480,), F32),
            pltpu.VMEM_SHARED((115200,), F32),
        ],
    )


def kernel(obs, pose_obs, maps_last, poses_last, agent_heights):
    depth = obs[0, 3].reshape(-1)
    pose = poses_last[0]
    rel = pose_obs[0]
    o_rad = pose[2] * float(np.pi / 180.0)
    yp = pose[1] + rel[1] * jnp.sin(o_rad) + rel[0] * jnp.cos(o_rad)
    xp = pose[0] + rel[1] * jnp.cos(o_rad) - rel[0] * jnp.sin(o_rad)
    o = pose[2] + rel[2] * 57.29577951308232
    o = jnp.fmod(o - 180.0, 360.0) + 180.0
    o = jnp.fmod(o + 180.0, 360.0) - 180.0
    current_poses = jnp.stack([xp, yp, o])[None]
    st0 = -(yp * 100.0 / 5.0 - 120.0) / 120.0
    st1 = -(xp * 100.0 / 5.0 - 120.0) / 120.0
    st2 = 90.0 - o
    tr = st2 * float(np.pi / 180.0)
    ctv = jnp.cos(tr)
    snv = jnp.sin(tr)
    ah = 88.0 * agent_heights[0]
    params = (jnp.stack([ah, ctv, snv, st0, st1]).astype(F32)[:, None]
              * jnp.ones((1, 16), F32)).reshape(-1)
    fp_flat, mp = _sc_call(depth, params, maps_last.reshape(-1))
    fp_map_pred = fp_flat.reshape(1, 1, 100, 100)
    return fp_map_pred, mp.reshape(1, 4, 240, 240), poses_last, current_poses


_sc_call = _make_sc_call()


# speedup vs baseline: 1.9934x; 1.1390x over previous
import numpy as np
import jax
import jax.numpy as jnp
from jax import lax
from jax.experimental import pallas as pl
from jax.experimental.pallas import tpu as pltpu
from jax.experimental.pallas import tpu_sc as plsc

F_CAM = float((640 / 2.0) / np.tan(np.deg2rad(79.0 / 2.0)))
COS90 = float(np.cos(np.pi / 2.0))
SIN90 = float(np.sin(np.pi / 2.0))
S239 = float(2.0 / 239.0)
INV_F = float(1.0 / F_CAM)
XF = float((1.0 / F_CAM) * 198.0)
PZC = float(0.2 * 87.0 / 88.0)
PZO = float(43.5 - 28.0 * 87.0 / 88.0)
F32 = jnp.float32
I32 = jnp.int32


def _floorparts(a):
    ti = a.astype(I32)
    fl = jnp.where(a < ti.astype(F32), ti - 1, ti)
    return fl, a - fl.astype(F32)


def _r16(x):
    u = plsc.bitcast(x, I32)
    rb = (u >> 16) & 1
    return plsc.bitcast((u + 32767 + rb) & (-65536), F32)


def _mask_f(m):
    return jnp.where(m, jnp.full((16,), 1.0, F32), jnp.full((16,), 0.0, F32))


def _sc_body(depth, params, maps, fp_out, mp_out,
             dbuf, planes, pv, slab, tmp, rotbuf, rotw, mapsbuf, sem,
             sh_all, sh_red, sh_rot):
    cid = lax.axis_index("c")
    sid = lax.axis_index("s")

    @pl.when(cid == 0)
    def _work():
        t = sid
        pltpu.sync_copy(params, pv)
        iota = lax.iota(I32, 16)
        ah = pv[pl.ds(0, 16)]
        ct = pv[pl.ds(16, 16)]
        sn = pv[pl.ds(32, 16)]
        tx = pv[pl.ds(48, 16)]
        ty = pv[pl.ds(64, 16)]
        zeros16 = jnp.full((16,), 0.0, F32)
        ct16 = _r16(ct)
        sn16 = _r16(sn)
        tx16 = _r16(tx)
        ty16 = _r16(ty)

        def _zero(i, c):
            planes[pl.ds(i * 16, 16)] = zeros16
            return c
        lax.fori_loop(0, 1280, _zero, 0)

        pltpu.sync_copy(depth.at[pl.ds(t * 19200, 19200)], dbuf)

        pzo = ah * PZC + PZO

        def _row_a(ri, c):
            r_img = t * 30 + ri
            pzf = (240.0 - r_img.astype(F32)) * (INV_F * 1000.0) * PZC
            roff = ri * 640

            def _vec_a(v, c2):
                c0 = v * 16
                draw = dbuf[pl.ds(roff + c0, 16)]
                xv = ((c0 + iota).astype(F32) - 320.0) * XF
                px = 49.5 - draw * 198.0
                py = xv * draw
                pz = draw * pzf + pzo
                ix, rx = _floorparts(px)
                iy, ry = _floorparts(py)
                iz, rz = _floorparts(pz)

                def _zin(z, lo, hi):
                    return _mask_f((z >= lo) & (z < hi))
                zall = (1.0 - rz) * _zin(iz, 0, 88) + rz * _zin(iz + 1, 0, 88)
                zag = (1.0 - rz) * _zin(iz, 21, 33) + rz * _zin(iz + 1, 21, 33)

                xok = ((ix >= 0) & (ix < 100), (ix >= -1) & (ix < 99))
                yok = ((iy >= 0) & (iy < 100), (iy >= -1) & (iy < 99))
                base = iy * 100 + ix
                for dx in (0, 1):
                    wx = rx if dx else 1.0 - rx
                    for dy in (0, 1):
                        wy = ry if dy else 1.0 - ry
                        m = xok[dx] & yok[dy]
                        idx = base + (dx + 100 * dy)
                        w = wx * wy
                        plsc.addupdate_scatter(planes, [idx], w * zag, mask=m)
                        plsc.addupdate_scatter(planes, [idx + 10000],
                                               w * zall, mask=m)
                return c2
            lax.fori_loop(0, 40, _vec_a, 0)
            return c
        lax.fori_loop(0, 30, _row_a, 0)

        pltpu.sync_copy(planes, sh_all.at[pl.ds(t * 20480, 20480)])
        plsc.subcore_barrier()
        off = t * 1280
        pltpu.sync_copy(sh_all.at[pl.ds(off, 1280)], slab)

        def _fire(k, c):
            pltpu.async_copy(sh_all.at[pl.ds(k * 20480 + off, 1280)],
                             tmp.at[pl.ds((k - 1) * 1280, 1280)], sem)
            return c
        lax.fori_loop(1, 16, _fire, 0)

        def _drain(k, c):
            pltpu.make_async_copy(
                sh_all.at[pl.ds(k * 20480 + off, 1280)],
                tmp.at[pl.ds((k - 1) * 1280, 1280)], sem).wait()
            return c
        lax.fori_loop(1, 16, _drain, 0)

        def _acc(k, c):
            toff = (k - 1) * 1280

            def _add(i, c2):
                s = pl.ds(i * 16, 16)
                slab[s] = slab[s] + tmp[pl.ds(toff + i * 16, 16)]
                return c2
            lax.fori_loop(0, 80, _add, 0)
            return c
        lax.fori_loop(1, 16, _acc, 0)
        pltpu.sync_copy(slab, sh_red.at[pl.ds(off, 1280)])
        plsc.subcore_barrier()
        pltpu.sync_copy(sh_red, planes)

        def _clip(i, c):
            s = pl.ds(i * 16, 16)
            planes[s] = jnp.minimum(jnp.maximum(planes[s], 0.0), 1.0)
            return c
        lax.fori_loop(0, 1280, _clip, 0)

        @pl.when(t == 0)
        def _fp():
            pltpu.sync_copy(planes.at[pl.ds(0, 10000)], fp_out)

        r0 = t * 15

        def _row_b(rl, c):
            r = r0 + rl
            gy16 = _r16(jnp.full((16,), -1.0, F32) + r.astype(F32) * S239)

            def _vec_b(v, c2):
                cc = v * 16 + iota
                gx16 = _r16(-1.0 + cc.astype(F32) * S239)
                x = (gx16 * ct16 - gy16 * sn16 + 1.0) * 119.5
                y = (gx16 * sn16 + gy16 * ct16 + 1.0) * 119.5
                x0, fxw = _floorparts(x)
                y0, fyw = _floorparts(y)
                acc0 = zeros16
                acc1 = zeros16
                for a in (0, 1):
                    xi = x0 + a if a else x0
                    wxv = fxw if a else 1.0 - fxw
                    for b in (0, 1):
                        yi = y0 + b if b else y0
                        wyv = fyw if b else 1.0 - fyw
                        m = ((yi >= 120) & (yi <= 219)
                             & (xi >= 70) & (xi <= 169))
                        pi = (jnp.clip(yi - 120, 0, 99) * 100
                              + jnp.clip(xi - 70, 0, 99))
                        w = wxv * wyv * _mask_f(m)
                        acc0 = acc0 + w * plsc.load_gather(planes, [pi])
                        acc1 = acc1 + w * plsc.load_gather(planes,
                                                           [pi + 10000])
                rotbuf[pl.ds(rl * 240 + v * 16, 16)] = acc0
                rotbuf[pl.ds(3600 + rl * 240 + v * 16, 16)] = acc1
                return c2
            lax.fori_loop(0, 15, _vec_b, 0)
            return c
        lax.fori_loop(0, 15, _row_b, 0)
        pltpu.sync_copy(rotbuf.at[pl.ds(0, 3600)],
                        sh_rot.at[pl.ds(r0 * 240, 3600)])
        pltpu.sync_copy(rotbuf.at[pl.ds(3600, 3600)],
                        sh_rot.at[pl.ds(57600 + r0 * 240, 3600)])
        plsc.subcore_barrier()

        y0v = (ty16 + _r16(jnp.full((16,), 0.0, F32)
                           + (-1.0 + r0.astype(F32) * S239)) + 1.0) * 119.5
        ys0 = jnp.min(y0v)
        si = ys0.astype(I32)
        si = jnp.where(ys0 < si.astype(F32), si - 1, si)
        start_c = jnp.clip(si - 1, 0, 240 - 24)
        pltpu.sync_copy(sh_rot.at[pl.ds(start_c * 240, 5760)],
                        rotw.at[pl.ds(0, 5760)])
        pltpu.sync_copy(sh_rot.at[pl.ds(57600 + start_c * 240, 5760)],
                        rotw.at[pl.ds(5760, 5760)])

        def _row_c(rl, c):
            r = r0 + rl
            gy16 = _r16(jnp.full((16,), -1.0, F32) + r.astype(F32) * S239)
            yrow = (gy16 + ty16 + 1.0) * 119.5
            y0i, fyw = _floorparts(yrow)

            def _vec_c(v, c2):
                cc = v * 16 + iota
                gx16 = _r16(-1.0 + cc.astype(F32) * S239)
                x = (gx16 + tx16 + 1.0) * 119.5
                x0i, fxw = _floorparts(x)
                acc0 = zeros16
                acc1 = zeros16
                for a in (0, 1):
                    xi = x0i + a if a else x0i
                    wxv = fxw if a else 1.0 - fxw
                    xm = (xi >= 0) & (xi <= 239)
                    xc = jnp.clip(xi, 0, 239)
                    for b in (0, 1):
                        yi = y0i + b if b else y0i
                        wyv = fyw if b else 1.0 - fyw
                        m = xm & (yi >= 0) & (yi <= 239)
                        rloc = jnp.clip(yi - start_c, 0, 23)
                        w = wxv * wyv * _mask_f(m)
                        gi = rloc * 240 + xc
                        acc0 = acc0 + w * plsc.load_gather(rotw, [gi])
                        acc1 = acc1 + w * plsc.load_gather(rotw, [gi + 5760])
                rotbuf[pl.ds(rl * 240 + v * 16, 16)] = acc0
                rotbuf[pl.ds(3600 + rl * 240 + v * 16, 16)] = acc1
                return c2
            lax.fori_loop(0, 15, _vec_c, 0)
            return c
        lax.fori_loop(0, 15, _row_c, 0)

        for ch in (0, 1):
            pltpu.sync_copy(maps.at[pl.ds(ch * 57600 + r0 * 240, 3600)],
                            mapsbuf)
            coff = ch * 3600

            def _mx(i, c2, coff=coff):
                sr = pl.ds(coff + i * 16, 16)
                sm = pl.ds(i * 16, 16)
                rotbuf[sr] = jnp.maximum(rotbuf[sr], mapsbuf[sm])
                return c2
            lax.fori_loop(0, 225, _mx, 0)
            pltpu.sync_copy(rotbuf.at[pl.ds(coff, 3600)],
                            mp_out.at[pl.ds(ch * 57600 + r0 * 240, 3600)])

        for ch in (2, 3):
            pltpu.sync_copy(maps.at[pl.ds(ch * 57600 + r0 * 240, 3600)],
                            mapsbuf)
            pltpu.sync_copy(mapsbuf,
                            mp_out.at[pl.ds(ch * 57600 + r0 * 240, 3600)])


def _make_sc_call():
    mesh = plsc.VectorSubcoreMesh(core_axis_name="c", subcore_axis_name="s")
    return pl.kernel(
        _sc_body,
        mesh=mesh,
        compiler_params=pltpu.CompilerParams(needs_layout_passes=False),
        out_type=(
            jax.ShapeDtypeStruct((10000,), F32),
            jax.ShapeDtypeStruct((230400,), F32),
        ),
        scratch_types=[
            pltpu.VMEM((19200,), F32),
            pltpu.VMEM((20480,), F32),
            pltpu.VMEM((80,), F32),
            pltpu.VMEM((1280,), F32),
            pltpu.VMEM((19200,), F32),
            pltpu.VMEM((7200,), F32),
            pltpu.VMEM((11520,), F32),
            pltpu.VMEM((3600,), F32),
            pltpu.SemaphoreType.DMA,
            pltpu.VMEM_SHARED((327680,), F32),
            pltpu.VMEM_SHARED((20480,), F32),
            pltpu.VMEM_SHARED((115200,), F32),
        ],
    )


def kernel(obs, pose_obs, maps_last, poses_last, agent_heights):
    depth = obs[0, 3].reshape(-1)
    pose = poses_last[0]
    rel = pose_obs[0]
    o_rad = pose[2] * float(np.pi / 180.0)
    yp = pose[1] + rel[1] * jnp.sin(o_rad) + rel[0] * jnp.cos(o_rad)
    xp = pose[0] + rel[1] * jnp.cos(o_rad) - rel[0] * jnp.sin(o_rad)
    o = pose[2] + rel[2] * 57.29577951308232
    o = jnp.fmod(o - 180.0, 360.0) + 180.0
    o = jnp.fmod(o + 180.0, 360.0) - 180.0
    current_poses = jnp.stack([xp, yp, o])[None]
    st0 = -(yp * 100.0 / 5.0 - 120.0) / 120.0
    st1 = -(xp * 100.0 / 5.0 - 120.0) / 120.0
    st2 = 90.0 - o
    tr = st2 * float(np.pi / 180.0)
    ctv = jnp.cos(tr)
    snv = jnp.sin(tr)
    ah = 88.0 * agent_heights[0]
    params = (jnp.stack([ah, ctv, snv, st0, st1]).astype(F32)[:, None]
              * jnp.ones((1, 16), F32)).reshape(-1)
    fp_flat, mp = _sc_call(depth, params, maps_last.reshape(-1))
    fp_map_pred = fp_flat.reshape(1, 1, 100, 100)
    return fp_map_pred, mp.reshape(1, 4, 240, 240), poses_last, current_poses


_sc_call = _make_sc_call()


# speedup vs baseline: 1.9941x; 1.0004x over previous
import numpy as np
import jax
import jax.numpy as jnp
from jax import lax
from jax.experimental import pallas as pl
from jax.experimental.pallas import tpu as pltpu
from jax.experimental.pallas import tpu_sc as plsc

F_CAM = float((640 / 2.0) / np.tan(np.deg2rad(79.0 / 2.0)))
COS90 = float(np.cos(np.pi / 2.0))
SIN90 = float(np.sin(np.pi / 2.0))
S239 = float(2.0 / 239.0)
INV_F = float(1.0 / F_CAM)
XF = float((1.0 / F_CAM) * 198.0)
PZC = float(0.2 * 87.0 / 88.0)
PZO = float(43.5 - 28.0 * 87.0 / 88.0)
F32 = jnp.float32
I32 = jnp.int32


def _floorparts(a):
    ti = a.astype(I32)
    fl = jnp.where(a < ti.astype(F32), ti - 1, ti)
    return fl, a - fl.astype(F32)


def _r16(x):
    u = plsc.bitcast(x, I32)
    rb = (u >> 16) & 1
    return plsc.bitcast((u + 32767 + rb) & (-65536), F32)


def _mask_f(m):
    return jnp.where(m, jnp.full((16,), 1.0, F32), jnp.full((16,), 0.0, F32))


def _sc_body(depth, params, maps, fp_out, mp_out,
             dbuf, planes, pv, slab, tmp, rotbuf, rotw, mapsbuf, sem,
             sh_all, sh_red, sh_rot):
    cid = lax.axis_index("c")
    sid = lax.axis_index("s")

    @pl.when(cid == 0)
    def _work():
        t = sid
        pltpu.sync_copy(params, pv)
        iota = lax.iota(I32, 16)
        ah = pv[pl.ds(0, 16)]
        ct = pv[pl.ds(16, 16)]
        sn = pv[pl.ds(32, 16)]
        tx = pv[pl.ds(48, 16)]
        ty = pv[pl.ds(64, 16)]
        zeros16 = jnp.full((16,), 0.0, F32)
        ct16 = _r16(ct)
        sn16 = _r16(sn)
        tx16 = _r16(tx)
        ty16 = _r16(ty)

        def _zero(i, c):
            planes[pl.ds(i * 16, 16)] = zeros16
            return c
        lax.fori_loop(0, 1280, _zero, 0)

        pltpu.sync_copy(depth.at[pl.ds(t * 19200, 19200)], dbuf)

        pzo = ah * PZC + PZO

        def _row_a(ri, c):
            r_img = t * 30 + ri
            pzf = (240.0 - r_img.astype(F32)) * (INV_F * 1000.0) * PZC
            roff = ri * 640

            def _vec_a(v, c2):
                c0 = v * 16
                draw = dbuf[pl.ds(roff + c0, 16)]
                xv = ((c0 + iota).astype(F32) - 320.0) * XF
                px = 49.5 - draw * 198.0
                py = xv * draw
                pz = draw * pzf + pzo
                ix, rx = _floorparts(px)
                iy, ry = _floorparts(py)
                iz, rz = _floorparts(pz)

                def _zin(z, lo, hi):
                    return _mask_f((z >= lo) & (z < hi))
                zall = (1.0 - rz) * _zin(iz, 0, 88) + rz * _zin(iz + 1, 0, 88)
                zag = (1.0 - rz) * _zin(iz, 21, 33) + rz * _zin(iz + 1, 21, 33)

                xok = ((ix >= 0) & (ix < 100), (ix >= -1) & (ix < 99))
                yok = ((iy >= 0) & (iy < 100), (iy >= -1) & (iy < 99))
                base = iy * 100 + ix
                for dx in (0, 1):
                    wx = rx if dx else 1.0 - rx
                    for dy in (0, 1):
                        wy = ry if dy else 1.0 - ry
                        m = xok[dx] & yok[dy]
                        idx = base + (dx + 100 * dy)
                        w = wx * wy
                        plsc.addupdate_scatter(planes, [idx], w * zag, mask=m)
                        plsc.addupdate_scatter(planes, [idx + 10000],
                                               w * zall, mask=m)
                return c2
            lax.fori_loop(0, 40, _vec_a, 0)
            return c
        lax.fori_loop(0, 30, _row_a, 0)

        pltpu.sync_copy(planes, sh_all.at[pl.ds(t * 20480, 20480)])
        plsc.subcore_barrier()
        off = t * 1280
        pltpu.sync_copy(sh_all.at[pl.ds(off, 1280)], slab)

        def _fire(k, c):
            pltpu.async_copy(sh_all.at[pl.ds(k * 20480 + off, 1280)],
                             tmp.at[pl.ds((k - 1) * 1280, 1280)], sem)
            return c
        lax.fori_loop(1, 16, _fire, 0)

        def _drain(k, c):
            pltpu.make_async_copy(
                sh_all.at[pl.ds(k * 20480 + off, 1280)],
                tmp.at[pl.ds((k - 1) * 1280, 1280)], sem).wait()
            return c
        lax.fori_loop(1, 16, _drain, 0)

        def _acc(k, c):
            toff = (k - 1) * 1280

            def _add(i, c2):
                s = pl.ds(i * 16, 16)
                slab[s] = slab[s] + tmp[pl.ds(toff + i * 16, 16)]
                return c2
            lax.fori_loop(0, 80, _add, 0)
            return c
        lax.fori_loop(1, 16, _acc, 0)
        pltpu.sync_copy(slab, sh_red.at[pl.ds(off, 1280)])
        plsc.subcore_barrier()
        pltpu.sync_copy(sh_red, planes)

        def _clip(i, c):
            s = pl.ds(i * 16, 16)
            planes[s] = jnp.minimum(jnp.maximum(planes[s], 0.0), 1.0)
            return c
        lax.fori_loop(0, 1280, _clip, 0)

        @pl.when(t == 0)
        def _fp():
            pltpu.sync_copy(planes.at[pl.ds(0, 10000)], fp_out)

        r0 = t * 15

        def _row_b(rl, c):
            r = r0 + rl
            gy16 = _r16(jnp.full((16,), -1.0, F32) + r.astype(F32) * S239)

            def _vec_b(v, c2):
                cc = v * 16 + iota
                gx16 = _r16(-1.0 + cc.astype(F32) * S239)
                x = (gx16 * ct16 - gy16 * sn16 + 1.0) * 119.5
                y = (gx16 * sn16 + gy16 * ct16 + 1.0) * 119.5
                x0, fxw = _floorparts(x)
                y0, fyw = _floorparts(y)
                xok = ((x0 >= 70) & (x0 <= 169), (x0 >= 69) & (x0 <= 168))
                yok = ((y0 >= 120) & (y0 <= 219), (y0 >= 119) & (y0 <= 218))
                xcl = (jnp.clip(x0 - 70, 0, 99), jnp.clip(x0 - 69, 0, 99))
                ycl = (jnp.clip(y0 - 120, 0, 99) * 100,
                       jnp.clip(y0 - 119, 0, 99) * 100)
                acc0 = zeros16
                acc1 = zeros16
                for a in (0, 1):
                    wxv = fxw if a else 1.0 - fxw
                    for b in (0, 1):
                        wyv = fyw if b else 1.0 - fyw
                        pi = ycl[b] + xcl[a]
                        w = wxv * wyv * _mask_f(xok[a] & yok[b])
                        acc0 = acc0 + w * plsc.load_gather(planes, [pi])
                        acc1 = acc1 + w * plsc.load_gather(planes,
                                                           [pi + 10000])
                rotbuf[pl.ds(rl * 240 + v * 16, 16)] = acc0
                rotbuf[pl.ds(3600 + rl * 240 + v * 16, 16)] = acc1
                return c2
            lax.fori_loop(0, 15, _vec_b, 0)
            return c
        lax.fori_loop(0, 15, _row_b, 0)
        pltpu.sync_copy(rotbuf.at[pl.ds(0, 3600)],
                        sh_rot.at[pl.ds(r0 * 240, 3600)])
        pltpu.sync_copy(rotbuf.at[pl.ds(3600, 3600)],
                        sh_rot.at[pl.ds(57600 + r0 * 240, 3600)])
        plsc.subcore_barrier()

        y0v = (ty16 + _r16(jnp.full((16,), 0.0, F32)
                           + (-1.0 + r0.astype(F32) * S239)) + 1.0) * 119.5
        ys0 = jnp.min(y0v)
        si = ys0.astype(I32)
        si = jnp.where(ys0 < si.astype(F32), si - 1, si)
        start_c = jnp.clip(si - 1, 0, 240 - 24)
        pltpu.sync_copy(sh_rot.at[pl.ds(start_c * 240, 5760)],
                        rotw.at[pl.ds(0, 5760)])
        pltpu.sync_copy(sh_rot.at[pl.ds(57600 + start_c * 240, 5760)],
                        rotw.at[pl.ds(5760, 5760)])

        def _row_c(rl, c):
            r = r0 + rl
            gy16 = _r16(jnp.full((16,), -1.0, F32) + r.astype(F32) * S239)
            yrow = (gy16 + ty16 + 1.0) * 119.5
            y0i, fyw = _floorparts(yrow)
            yok = ((y0i >= 0) & (y0i <= 239), (y0i >= -1) & (y0i <= 238))
            rcl = (jnp.clip(y0i - start_c, 0, 23) * 240,
                   jnp.clip(y0i + 1 - start_c, 0, 23) * 240)

            def _vec_c(v, c2):
                cc = v * 16 + iota
                gx16 = _r16(-1.0 + cc.astype(F32) * S239)
                x = (gx16 + tx16 + 1.0) * 119.5
                x0i, fxw = _floorparts(x)
                xok = ((x0i >= 0) & (x0i <= 239),
                       (x0i >= -1) & (x0i <= 238))
                xcl = (jnp.clip(x0i, 0, 239), jnp.clip(x0i + 1, 0, 239))
                acc0 = zeros16
                acc1 = zeros16
                for a in (0, 1):
                    wxv = fxw if a else 1.0 - fxw
                    for b in (0, 1):
                        wyv = fyw if b else 1.0 - fyw
                        w = wxv * wyv * _mask_f(xok[a] & yok[b])
                        gi = rcl[b] + xcl[a]
                        acc0 = acc0 + w * plsc.load_gather(rotw, [gi])
                        acc1 = acc1 + w * plsc.load_gather(rotw, [gi + 5760])
                rotbuf[pl.ds(rl * 240 + v * 16, 16)] = acc0
                rotbuf[pl.ds(3600 + rl * 240 + v * 16, 16)] = acc1
                return c2
            lax.fori_loop(0, 15, _vec_c, 0)
            return c
        lax.fori_loop(0, 15, _row_c, 0)

        for ch in (0, 1):
            pltpu.sync_copy(maps.at[pl.ds(ch * 57600 + r0 * 240, 3600)],
                            mapsbuf)
            coff = ch * 3600

            def _mx(i, c2, coff=coff):
                sr = pl.ds(coff + i * 16, 16)
                sm = pl.ds(i * 16, 16)
                rotbuf[sr] = jnp.maximum(rotbuf[sr], mapsbuf[sm])
                return c2
            lax.fori_loop(0, 225, _mx, 0)
            pltpu.sync_copy(rotbuf.at[pl.ds(coff, 3600)],
                            mp_out.at[pl.ds(ch * 57600 + r0 * 240, 3600)])

        for ch in (2, 3):
            pltpu.sync_copy(maps.at[pl.ds(ch * 57600 + r0 * 240, 3600)],
                            mapsbuf)
            pltpu.sync_copy(mapsbuf,
                            mp_out.at[pl.ds(ch * 57600 + r0 * 240, 3600)])


def _make_sc_call():
    mesh = plsc.VectorSubcoreMesh(core_axis_name="c", subcore_axis_name="s")
    return pl.kernel(
        _sc_body,
        mesh=mesh,
        compiler_params=pltpu.CompilerParams(needs_layout_passes=False),
        out_type=(
            jax.ShapeDtypeStruct((10000,), F32),
            jax.ShapeDtypeStruct((230400,), F32),
        ),
        scratch_types=[
            pltpu.VMEM((19200,), F32),
            pltpu.VMEM((20480,), F32),
            pltpu.VMEM((80,), F32),
            pltpu.VMEM((1280,), F32),
            pltpu.VMEM((19200,), F32),
            pltpu.VMEM((7200,), F32),
            pltpu.VMEM((11520,), F32),
            pltpu.VMEM((3600,), F32),
            pltpu.SemaphoreType.DMA,
            pltpu.VMEM_SHARED((327680,), F32),
            pltpu.VMEM_SHARED((20480,), F32),
            pltpu.VMEM_SHARED((115200,), F32),
        ],
    )


def kernel(obs, pose_obs, maps_last, poses_last, agent_heights):
    depth = obs[0, 3].reshape(-1)
    pose = poses_last[0]
    rel = pose_obs[0]
    o_rad = pose[2] * float(np.pi / 180.0)
    yp = pose[1] + rel[1] * jnp.sin(o_rad) + rel[0] * jnp.cos(o_rad)
    xp = pose[0] + rel[1] * jnp.cos(o_rad) - rel[0] * jnp.sin(o_rad)
    o = pose[2] + rel[2] * 57.29577951308232
    o = jnp.fmod(o - 180.0, 360.0) + 180.0
    o = jnp.fmod(o + 180.0, 360.0) - 180.0
    current_poses = jnp.stack([xp, yp, o])[None]
    st0 = -(yp * 100.0 / 5.0 - 120.0) / 120.0
    st1 = -(xp * 100.0 / 5.0 - 120.0) / 120.0
    st2 = 90.0 - o
    tr = st2 * float(np.pi / 180.0)
    ctv = jnp.cos(tr)
    snv = jnp.sin(tr)
    ah = 88.0 * agent_heights[0]
    params = (jnp.stack([ah, ctv, snv, st0, st1]).astype(F32)[:, None]
              * jnp.ones((1, 16), F32)).reshape(-1)
    fp_flat, mp = _sc_call(depth, params, maps_last.reshape(-1))
    fp_map_pred = fp_flat.reshape(1, 1, 100, 100)
    return fp_map_pred, mp.reshape(1, 4, 240, 240), poses_last, current_poses


_sc_call = _make_sc_call()
